# baseline (speedup 1.0000x reference)
import jax
import jax.numpy as jnp
from jax import lax
from jax.experimental import pallas as pl
from jax.experimental.pallas import tpu as pltpu
from jax.experimental.pallas import tpu_sc as plsc

_N = 1048576
_W = 128
_ROWS = _N // _W
_K = 256
_G = _ROWS // _K

_NW = 32
_PW = _N // _NW
_C = 4096
_NCH = _PW // _C
_NB = 2
_LN2 = 0.6931471805599453


def _logf(q):
    acc = jnp.zeros((16,), jnp.float32)
    for thresh, scale, shift in (
        (2.0 ** -8, 256.0, 8.0 * _LN2),
        (2.0 ** -4, 16.0, 4.0 * _LN2),
        (0.25, 4.0, 2.0 * _LN2),
        (0.5, 2.0, _LN2),
    ):
        c = q < thresh
        q = jnp.where(c, q * scale, q)
        acc = jnp.where(c, acc - shift, acc)
    t = (q - 1.0) / (q + 1.0)
    t2 = t * t
    poly = 1.0 + t2 * (1.0 / 3.0 + t2 * (0.2 + t2 * (1.0 / 7.0 + t2 / 9.0)))
    return acc + 2.0 * t * poly


def _sc_body(gt_h, p_h, tr_h, pr_h, out_h,
             gt_b, p_b, tr_b, pr_b, part_b, sem0, sem1):
    cid = lax.axis_index("c")
    sid = lax.axis_index("s")
    wid = sid * 2 + cid
    base0 = wid * _PW
    sems = [sem0, sem1]

    def _copies(chunk, b):
        base = base0 + chunk * _C
        return [
            (gt_h.at[pl.ds(base, _C)], gt_b.at[b]),
            (p_h.at[pl.ds(base, _C)], p_b.at[b]),
            (tr_h.at[pl.ds(base, _C)], tr_b.at[b]),
            (pr_h.at[pl.ds(base, _C)], pr_b.at[b]),
        ]

    def issue(chunk, b):
        for src, dst in _copies(chunk, b):
            pltpu.async_copy(src, dst, sems[b])

    def waitall(chunk, b):
        for src, dst in _copies(chunk, b):
            pltpu.make_async_copy(src, dst, sems[b]).wait()

    def compute_chunk(b, accs):
        gt_r = gt_b.at[b]
        p_r = p_b.at[b]
        tr_r = tr_b.at[b]
        pr_r = pr_b.at[b]

        def vec_body(u, accs):
            a1, a2, a5, a6 = accs
            s = pl.ds(u * 16, 16)
            gtv = gt_r[s]
            pv = p_r[s]
            trv = tr_r[s]
            prv = pr_r[s]
            one = jnp.float32(1.0)
            zero = jnp.float32(0.0)

            mb = jnp.abs(gtv)
            mf = jnp.where(gtv >= 0.0, one, zero)
            q = jnp.where(gtv == 1.0, pv, one - pv)
            a1 = a1 - mf * _logf(q)
            a2 = a2 + mf

            mr = jnp.where(trv >= 0.0, mb, zero)
            qr = jnp.where(trv == 1.0, prv, one - prv)
            a5 = a5 - mr * _logf(qr)
            a6 = a6 + mr
            return (a1, a2, a5, a6)

        return lax.fori_loop(0, _C // 16, vec_body, accs, unroll=8)

    for b in range(_NB):
        issue(b, b)

    z = jnp.zeros((16,), jnp.float32)
    accs = (z, z, z, z)

    def chunk_body(g2, accs):
        for b in range(_NB):
            chunk = g2 * _NB + b
            waitall(chunk, b)
            accs = compute_chunk(b, accs)
            issue(chunk + _NB, b)
        return accs

    accs = lax.fori_loop(0, _NCH // _NB - 1, chunk_body, accs)
    for b in range(_NB):
        chunk = _NCH - _NB + b
        waitall(chunk, b)
        accs = compute_chunk(b, accs)

    part_b[...] = jnp.zeros((_W,), jnp.float32)
    for k in range(4):
        part_b[pl.ds(16 * k, 16)] = accs[k]
    pltpu.sync_copy(part_b, out_h.at[wid])


def _box_body(gt_ref, bt_ref, bp_ref, out_ref, acc_ref, s_ref):
    i = pl.program_id(0)

    @pl.when(i == 0)
    def _init():
        acc_ref[...] = jnp.zeros_like(acc_ref)
        ri = jax.lax.broadcasted_iota(jnp.int32, (4 * _K, _K), 0)
        ki = jax.lax.broadcasted_iota(jnp.int32, (4 * _K, _K), 1)
        s_ref[...] = jnp.where(ri // 4 == ki, 1.0, 0.0).astype(jnp.bfloat16)

    mb = jnp.abs(gt_ref[...])
    d = bp_ref[...] - bt_ref[...]
    sq = d * d
    e = jax.lax.dot(s_ref[...], mb.astype(jnp.bfloat16),
                    preferred_element_type=jnp.float32)
    acc_ref[0] += jnp.sum((sq * e).reshape(-1, 8, _W), axis=0)
    acc_ref[1] += jnp.sum(mb.reshape(-1, 8, _W), axis=0)

    @pl.when(i == _G - 1)
    def _fin():
        out_ref[0, 0] = jnp.sum(acc_ref[0])
        out_ref[0, 1] = jnp.sum(acc_ref[1])


def _combine_body(x_ref, b_ref, o_ref):
    x = x_ref[...]
    s = [jnp.sum(x[:, 16 * k:16 * (k + 1)]) for k in range(4)]
    face = s[0] / s[1]
    box = b_ref[0, 0] / (b_ref[0, 1] * 4.0) * 0.5
    rig = s[2] / s[3] * 0.5
    o_ref[0, 0] = face + box + rig


def _flat_box(b):
    return b.reshape(_ROWS, _W, 4).transpose(0, 2, 1).reshape(4 * _ROWS, _W)


def kernel(gt_label, pred_label, box_target, box_pred, target_rig, pred_rig):
    gt2 = gt_label.reshape(_ROWS, _W)
    bt = _flat_box(box_target)
    bp = _flat_box(box_pred)
    p1 = pred_label.reshape(_N)

    mesh = plsc.VectorSubcoreMesh(core_axis_name="c", subcore_axis_name="s")
    sc_partials = pl.kernel(
        _sc_body,
        mesh=mesh,
        out_type=jax.ShapeDtypeStruct((_NW, _W), jnp.float32),
        scratch_types=[
            pltpu.VMEM((_NB, _C), jnp.float32),
            pltpu.VMEM((_NB, _C), jnp.float32),
            pltpu.VMEM((_NB, _C), jnp.float32),
            pltpu.VMEM((_NB, _C), jnp.float32),
            pltpu.VMEM((_W,), jnp.float32),
            pltpu.SemaphoreType.DMA,
            pltpu.SemaphoreType.DMA,
        ],
    )(gt_label, p1, target_rig, pred_rig)

    box_partials = pl.pallas_call(
        _box_body,
        grid=(_G,),
        in_specs=[
            pl.BlockSpec((_K, _W), lambda i: (i, 0)),
            pl.BlockSpec((4 * _K, _W), lambda i: (i, 0)),
            pl.BlockSpec((4 * _K, _W), lambda i: (i, 0)),
        ],
        out_specs=pl.BlockSpec(memory_space=pltpu.SMEM),
        out_shape=jax.ShapeDtypeStruct((1, 2), jnp.float32),
        scratch_shapes=[
            pltpu.VMEM((2, 8, _W), jnp.float32),
            pltpu.VMEM((4 * _K, _K), jnp.bfloat16),
        ],
        compiler_params=pltpu.CompilerParams(
            dimension_semantics=("arbitrary",),
        ),
    )(gt2, bt, bp)

    out = pl.pallas_call(
        _combine_body,
        in_specs=[
            pl.BlockSpec((_NW, _W), lambda: (0, 0)),
            pl.BlockSpec(memory_space=pltpu.SMEM),
        ],
        out_specs=pl.BlockSpec(memory_space=pltpu.SMEM),
        out_shape=jax.ShapeDtypeStruct((1, 1), jnp.float32),
    )(sc_partials, box_partials)
    return out[0, 0]


# speedup vs baseline: 3.2817x; 3.2817x over previous
import jax
import jax.numpy as jnp
from jax.experimental import pallas as pl
from jax.experimental.pallas import tpu as pltpu

_N = 1048576
_W = 128
_ROWS = _N // _W
_K = 512
_G = _ROWS // _K


def _fold(x):
    return jnp.sum(x.reshape(-1, 8, _W), axis=0)


def _body(gt_ref, p_ref, bt_ref, bp_ref, tr_ref, pr_ref, out_ref,
          acc_ref, s_ref):
    i = pl.program_id(0)

    @pl.when(i == 0)
    def _init():
        acc_ref[...] = jnp.zeros_like(acc_ref)
        ri = jax.lax.broadcasted_iota(jnp.int32, (4 * _K, _K), 0)
        ki = jax.lax.broadcasted_iota(jnp.int32, (4 * _K, _K), 1)
        s_ref[...] = jnp.where(ri // 4 == ki, 1.0, 0.0).astype(jnp.bfloat16)

    gt = gt_ref[...]
    p = p_ref[...]
    tr = tr_ref[...]
    pr = pr_ref[...]

    one = jnp.float32(1.0)
    zero = jnp.float32(0.0)

    mask_f = gt >= 0.0
    mf = jnp.where(mask_f, one, zero)
    q = jnp.where(gt == 1.0, p, one - p)
    bce_f = mf * jnp.log(q)

    mb = jnp.abs(gt)
    mr = jnp.where(tr >= 0.0, mb, zero)
    qr = jnp.where(tr == 1.0, pr, one - pr)
    bce_r = mr * jnp.log(qr)

    d = bp_ref[...] - bt_ref[...]
    sq = d * d
    e = jax.lax.dot(s_ref[...], mb.astype(jnp.bfloat16),
                    preferred_element_type=jnp.float32)

    acc_ref[0] -= _fold(bce_f)
    acc_ref[1] += _fold(mf)
    acc_ref[2] += _fold(sq * e)
    acc_ref[3] += _fold(mb)
    acc_ref[4] -= _fold(bce_r)
    acc_ref[5] += _fold(mr)

    @pl.when(i == _G - 1)
    def _fin():
        s = [jnp.sum(acc_ref[k]) for k in range(6)]
        face = s[0] / s[1]
        box = s[2] / (s[3] * 4.0) * 0.5
        rig = s[4] / s[5] * 0.5
        out_ref[0, 0] = face + box + rig


def _flat_box(b):
    return b.reshape(_ROWS, _W, 4).transpose(0, 2, 1).reshape(4 * _ROWS, _W)


def kernel(gt_label, pred_label, box_target, box_pred, target_rig, pred_rig):
    gt = gt_label.reshape(_ROWS, _W)
    p = pred_label.reshape(_ROWS, _W)
    bt = _flat_box(box_target)
    bp = _flat_box(box_pred)
    tr = target_rig.reshape(_ROWS, _W)
    pr = pred_rig.reshape(_ROWS, _W)

    out = pl.pallas_call(
        _body,
        grid=(_G,),
        in_specs=[
            pl.BlockSpec((_K, _W), lambda i: (i, 0)),
            pl.BlockSpec((_K, _W), lambda i: (i, 0)),
            pl.BlockSpec((4 * _K, _W), lambda i: (i, 0)),
            pl.BlockSpec((4 * _K, _W), lambda i: (i, 0)),
            pl.BlockSpec((_K, _W), lambda i: (i, 0)),
            pl.BlockSpec((_K, _W), lambda i: (i, 0)),
        ],
        out_specs=pl.BlockSpec(memory_space=pltpu.SMEM),
        out_shape=jax.ShapeDtypeStruct((1, 1), jnp.float32),
        scratch_shapes=[
            pltpu.VMEM((6, 8, _W), jnp.float32),
            pltpu.VMEM((4 * _K, _K), jnp.bfloat16),
        ],
        compiler_params=pltpu.CompilerParams(
            dimension_semantics=("arbitrary",),
        ),
    )(gt, p, bt, bp, tr, pr)
    return out[0, 0]
